# baseline (speedup 1.0000x reference)
import functools

import jax
import jax.numpy as jnp
from jax import lax
from jax.experimental import pallas as pl
from jax.experimental.pallas import tpu as pltpu
from jax.experimental.pallas import tpu_sc as plsc

N = 10000
D = 128
H = 128
NC = 2
NS = 16
NW = NC * NS
C = 80
NB = 4
NP = 10240
RPW = NP // NS

_MESH = plsc.VectorSubcoreMesh(
    core_axis_name="c", subcore_axis_name="s", num_cores=NC, num_subcores=NS
)


def _num_groups(e: int) -> int:
    ng = -(-e // (NW * C * NB))
    while ng < 2 or (ng - 2) % 3 != 0:
        ng += 1
    return ng


def _zero_spmem(zblk_hbm, rows0, sh, sid, sem):
    pltpu.sync_copy(zblk_hbm, rows0)
    descs = [
        pltpu.async_copy(rows0, sh.at[pl.ds(sid * RPW + r * C, C)], sem)
        for r in range(RPW // C)
    ]
    for d in descs:
        d.wait()


def _copy_out(sh, rows, out_hbm, cid, sid, sems):
    nb = len(rows)
    nch = RPW // C
    ins = {}
    for r in range(min(nb, nch)):
        off = sid * RPW + r * C
        ins[r] = pltpu.async_copy(sh.at[pl.ds(off, C)], rows[r % nb],
                                  sems[r % nb])
    outs = {}
    for r in range(nch):
        off = sid * RPW + r * C
        ins[r].wait()
        outs[r] = pltpu.async_copy(rows[r % nb], out_hbm.at[cid, pl.ds(off, C)],
                                   sems[r % nb])
        nxt = r + nb
        if nxt < nch:
            outs[r].wait()
            noff = sid * RPW + nxt * C
            ins[nxt] = pltpu.async_copy(sh.at[pl.ds(noff, C)], rows[nxt % nb],
                                        sems[nxt % nb])
    for r in range(max(0, nch - nb), nch):
        outs[r].wait()


def _make_sc_agg(ng: int):
    num_chunks = ng * NB

    @functools.partial(
        pl.kernel,
        out_type=jax.ShapeDtypeStruct((NC, NP, D), jnp.float32),
        mesh=_MESH,
        scratch_types=(
            [pltpu.VMEM((C,), jnp.int32) for _ in range(3 * NB)]
            + [pltpu.VMEM((C,), jnp.int32) for _ in range(3 * NB)]
            + [pltpu.VMEM((C, D), jnp.float32) for _ in range(NB)]
            + [pltpu.VMEM_SHARED((NP, D), jnp.float32)]
            + [pltpu.SemaphoreType.DMA for _ in range(3 * NB)]
            + [pltpu.SemaphoreType.DMA for _ in range(NB)]
            + [pltpu.SemaphoreType.DMA for _ in range(NB)]
        ),
    )
    def sc_agg(h_hbm, srcs_hbm, dsts_hbm, zblk_hbm, agg_out, *scratch):
        src_v = scratch[0:3 * NB]
        dst_v = scratch[3 * NB:6 * NB]
        rows_v = scratch[6 * NB:7 * NB]
        agg_sh = scratch[7 * NB]
        sem_i = scratch[7 * NB + 1:10 * NB + 1]
        sem_g = scratch[10 * NB + 1:11 * NB + 1]
        sem_s = scratch[11 * NB + 1:12 * NB + 1]
        cid = lax.axis_index("c")
        sid = lax.axis_index("s")
        wid = cid * NS + sid
        wbase = wid * num_chunks

        _zero_spmem(zblk_hbm, rows_v[0], agg_sh, sid, sem_g[0])
        plsc.subcore_barrier()

        def fire_idx(g, q):
            for b in range(NB):
                row = wbase + g * NB + b
                pltpu.async_copy(srcs_hbm.at[row], src_v[q * NB + b],
                                 sem_i[q * NB + b])
                pltpu.async_copy(dsts_hbm.at[row], dst_v[q * NB + b],
                                 sem_i[q * NB + b])

        def drain_s(q):
            for b in range(NB):
                pltpu.make_async_copy(rows_v[b],
                                      agg_sh.at[dst_v[q * NB + b]],
                                      sem_s[b]).wait()

        def group(g, q, drain, prefetch):
            if prefetch:
                fire_idx(g + 1, (q + 1) % 3)
            if drain:
                drain_s((q + 2) % 3)
            dg = []
            for b in range(NB):
                i = q * NB + b
                pltpu.make_async_copy(srcs_hbm.at[0], src_v[i],
                                      sem_i[i]).wait()
                pltpu.make_async_copy(dsts_hbm.at[0], dst_v[i],
                                      sem_i[i]).wait()
                dg.append(pltpu.async_copy(h_hbm.at[src_v[i]], rows_v[b],
                                           sem_g[b]))
            for b in range(NB):
                dg[b].wait()
                pltpu.async_copy(rows_v[b], agg_sh.at[dst_v[q * NB + b]],
                                 sem_s[b], add=True)

        fire_idx(0, 0)
        group(0, 0, drain=False, prefetch=True)

        def body(t, _):
            g = 3 * t + 1
            group(g, 1, drain=True, prefetch=True)
            group(g + 1, 2, drain=True, prefetch=True)
            group(g + 2, 0, drain=True, prefetch=True)
            return _

        lax.fori_loop(0, (ng - 2) // 3, body, None)

        group(ng - 1, (ng - 1) % 3, drain=True, prefetch=False)
        drain_s((ng - 1) % 3)

        plsc.subcore_barrier()
        _copy_out(agg_sh, rows_v, agg_out, cid, sid, sem_g)

    return sc_agg


def _make_sc_deg(ng: int):
    num_chunks = ng * NB

    @functools.partial(
        pl.kernel,
        out_type=jax.ShapeDtypeStruct((NC, NP, D), jnp.float32),
        mesh=_MESH,
        scratch_types=(
            [pltpu.VMEM((C,), jnp.int32) for _ in range(3 * NB)]
            + [pltpu.VMEM((C, D), jnp.float32) for _ in range(2)]
            + [pltpu.VMEM((C, D), jnp.float32)]
            + [pltpu.VMEM_SHARED((NP, D), jnp.float32)]
            + [pltpu.SemaphoreType.DMA for _ in range(3 * NB)]
            + [pltpu.SemaphoreType.DMA for _ in range(NB)]
        ),
    )
    def sc_deg(dsts_hbm, zblk_hbm, oblk_hbm, deg_out, *scratch):
        dst_v = scratch[0:3 * NB]
        bounce = scratch[3 * NB:3 * NB + 2]
        ones_v = scratch[3 * NB + 2]
        deg_sh = scratch[3 * NB + 3]
        sem_i = scratch[3 * NB + 4:6 * NB + 4]
        sem_s = scratch[6 * NB + 4:7 * NB + 4]
        cid = lax.axis_index("c")
        sid = lax.axis_index("s")
        wid = cid * NS + sid
        wbase = wid * num_chunks

        _zero_spmem(zblk_hbm, bounce[0], deg_sh, sid, sem_s[0])
        pltpu.sync_copy(oblk_hbm, ones_v)
        plsc.subcore_barrier()

        def fire_idx(g, q):
            for b in range(NB):
                row = wbase + g * NB + b
                pltpu.async_copy(dsts_hbm.at[row], dst_v[q * NB + b],
                                 sem_i[q * NB + b])

        def drain_s(q):
            for b in range(NB):
                pltpu.make_async_copy(ones_v, deg_sh.at[dst_v[q * NB + b]],
                                      sem_s[b]).wait()

        def group(g, q, drain, prefetch):
            if prefetch:
                fire_idx(g + 1, (q + 1) % 3)
            if drain:
                drain_s((q + 2) % 3)
            for b in range(NB):
                i = q * NB + b
                pltpu.make_async_copy(dsts_hbm.at[0], dst_v[i],
                                      sem_i[i]).wait()
                pltpu.async_copy(ones_v, deg_sh.at[dst_v[i]],
                                 sem_s[b], add=True)

        fire_idx(0, 0)
        group(0, 0, drain=False, prefetch=True)

        def body(t, _):
            g = 3 * t + 1
            group(g, 1, drain=True, prefetch=True)
            group(g + 1, 2, drain=True, prefetch=True)
            group(g + 2, 0, drain=True, prefetch=True)
            return _

        lax.fori_loop(0, (ng - 2) // 3, body, None)

        group(ng - 1, (ng - 1) % 3, drain=True, prefetch=False)
        drain_s((ng - 1) % 3)

        plsc.subcore_barrier()
        _copy_out(deg_sh, bounce, deg_out, cid, sid, sem_s)

    return sc_deg


def _make_tc_layer(final: bool, bn: int = 1000):
    def body(x_ref, p_ref, deg_ref, ws_ref, wn_ref, b_ref, o_ref):
        xb = x_ref[...]
        pb = p_ref[0] + p_ref[1]
        deg = deg_ref[0, :, 0:1] + deg_ref[1, :, 0:1]
        hn = pb / jnp.maximum(deg, 1.0)
        acc = lax.dot_general(xb, ws_ref[...], (((1,), (1,)), ((), ())),
                              preferred_element_type=jnp.float32)
        acc = acc + lax.dot_general(hn, wn_ref[...], (((1,), (1,)), ((), ())),
                                    preferred_element_type=jnp.float32)
        acc = acc + b_ref[...]
        acc = jnp.maximum(acc, 0.0)
        if final:
            nrm = jnp.sqrt(jnp.sum(acc * acc, axis=1, keepdims=True))
            acc = acc / jnp.maximum(nrm, 1e-12)
        o_ref[...] = acc

    grid = N // bn
    return pl.pallas_call(
        body,
        grid=(grid,),
        in_specs=[
            pl.BlockSpec((bn, D), lambda i: (i, 0)),
            pl.BlockSpec((NC, bn, D), lambda i: (0, i, 0)),
            pl.BlockSpec((NC, bn, D), lambda i: (0, i, 0)),
            pl.BlockSpec((H, D), lambda i: (0, 0)),
            pl.BlockSpec((H, D), lambda i: (0, 0)),
            pl.BlockSpec((1, H), lambda i: (0, 0)),
        ],
        out_specs=pl.BlockSpec((bn, H), lambda i: (i, 0)),
        out_shape=jax.ShapeDtypeStruct((N, H), jnp.float32),
    )


def kernel(x, edge_index, W_self1, W_neigh1, b1, W_self2, W_neigh2, b2):
    e = edge_index.shape[1]
    ng = _num_groups(e)
    num_chunks = ng * NB
    pad = NW * num_chunks * C - e

    src = edge_index[0]
    dst = edge_index[1]
    pad_idx = jnp.arange(pad, dtype=jnp.int32)
    srcs = jnp.concatenate([src, (pad_idx * 1315) % N]).reshape(
        NW * num_chunks, C)
    dsts = jnp.concatenate([dst, N + pad_idx % 192]).reshape(
        NW * num_chunks, C)

    zblk = jnp.zeros((C, D), jnp.float32)
    oblk = jnp.ones((C, D), jnp.float32)

    sc_agg = _make_sc_agg(ng)
    sc_deg = _make_sc_deg(ng)
    tc_layer1 = _make_tc_layer(final=False)
    tc_layer2 = _make_tc_layer(final=True)

    deg = sc_deg(dsts, zblk, oblk)
    agg1 = sc_agg(x, srcs, dsts, zblk)
    h1 = tc_layer1(x, agg1, deg, W_self1, W_neigh1, b1.reshape(1, H))
    agg2 = sc_agg(h1, srcs, dsts, zblk)
    out = tc_layer2(h1, agg2, deg, W_self2, W_neigh2, b2.reshape(1, H))
    return out


# speedup vs baseline: 1.0208x; 1.0208x over previous
import functools

import jax
import jax.numpy as jnp
from jax import lax
from jax.experimental import pallas as pl
from jax.experimental.pallas import tpu as pltpu
from jax.experimental.pallas import tpu_sc as plsc

N = 10000
D = 128
H = 128
NC = 2
NS = 16
NW = NC * NS
C = 64
NB = 5
NP = 10240
RPW = NP // NS

_MESH = plsc.VectorSubcoreMesh(
    core_axis_name="c", subcore_axis_name="s", num_cores=NC, num_subcores=NS
)


def _num_groups(e: int) -> int:
    ng = -(-e // (NW * C * NB))
    while ng < 2 or (ng - 2) % 3 != 0:
        ng += 1
    return ng


def _zero_spmem(zblk_hbm, rows0, sh, sid, sem):
    pltpu.sync_copy(zblk_hbm, rows0)
    descs = [
        pltpu.async_copy(rows0, sh.at[pl.ds(sid * RPW + r * C, C)], sem)
        for r in range(RPW // C)
    ]
    for d in descs:
        d.wait()


def _copy_out(sh, rows, out_hbm, cid, sid, sems):
    nb = len(rows)
    nch = RPW // C
    ins = {}
    for r in range(min(nb, nch)):
        off = sid * RPW + r * C
        ins[r] = pltpu.async_copy(sh.at[pl.ds(off, C)], rows[r % nb],
                                  sems[r % nb])
    outs = {}
    for r in range(nch):
        off = sid * RPW + r * C
        ins[r].wait()
        outs[r] = pltpu.async_copy(rows[r % nb], out_hbm.at[cid, pl.ds(off, C)],
                                   sems[r % nb])
        nxt = r + nb
        if nxt < nch:
            outs[r].wait()
            noff = sid * RPW + nxt * C
            ins[nxt] = pltpu.async_copy(sh.at[pl.ds(noff, C)], rows[nxt % nb],
                                        sems[nxt % nb])
    for r in range(max(0, nch - nb), nch):
        outs[r].wait()


def _make_sc_agg(ng: int):
    num_chunks = ng * NB

    @functools.partial(
        pl.kernel,
        out_type=jax.ShapeDtypeStruct((NC, NP, D), jnp.float32),
        mesh=_MESH,
        scratch_types=(
            [pltpu.VMEM((C,), jnp.int32) for _ in range(3 * NB)]
            + [pltpu.VMEM((C,), jnp.int32) for _ in range(3 * NB)]
            + [pltpu.VMEM((C, D), jnp.float32) for _ in range(NB)]
            + [pltpu.VMEM_SHARED((NP, D), jnp.float32)]
            + [pltpu.SemaphoreType.DMA for _ in range(3 * NB)]
            + [pltpu.SemaphoreType.DMA for _ in range(NB)]
            + [pltpu.SemaphoreType.DMA for _ in range(NB)]
        ),
    )
    def sc_agg(h_hbm, srcs_hbm, dsts_hbm, zblk_hbm, agg_out, *scratch):
        src_v = scratch[0:3 * NB]
        dst_v = scratch[3 * NB:6 * NB]
        rows_v = scratch[6 * NB:7 * NB]
        agg_sh = scratch[7 * NB]
        sem_i = scratch[7 * NB + 1:10 * NB + 1]
        sem_g = scratch[10 * NB + 1:11 * NB + 1]
        sem_s = scratch[11 * NB + 1:12 * NB + 1]
        cid = lax.axis_index("c")
        sid = lax.axis_index("s")
        wid = cid * NS + sid
        wbase = wid * num_chunks

        _zero_spmem(zblk_hbm, rows_v[0], agg_sh, sid, sem_g[0])
        plsc.subcore_barrier()

        def fire_idx(g, q):
            for b in range(NB):
                row = wbase + g * NB + b
                pltpu.async_copy(srcs_hbm.at[row], src_v[q * NB + b],
                                 sem_i[q * NB + b])
                pltpu.async_copy(dsts_hbm.at[row], dst_v[q * NB + b],
                                 sem_i[q * NB + b])

        def drain_s(q):
            for b in range(NB):
                pltpu.make_async_copy(rows_v[b],
                                      agg_sh.at[dst_v[q * NB + b]],
                                      sem_s[b]).wait()

        def group(g, q, drain, prefetch):
            if prefetch:
                fire_idx(g + 1, (q + 1) % 3)
            if drain:
                drain_s((q + 2) % 3)
            dg = []
            for b in range(NB):
                i = q * NB + b
                pltpu.make_async_copy(srcs_hbm.at[0], src_v[i],
                                      sem_i[i]).wait()
                pltpu.make_async_copy(dsts_hbm.at[0], dst_v[i],
                                      sem_i[i]).wait()
                dg.append(pltpu.async_copy(h_hbm.at[src_v[i]], rows_v[b],
                                           sem_g[b]))
            for b in range(NB):
                dg[b].wait()
                pltpu.async_copy(rows_v[b], agg_sh.at[dst_v[q * NB + b]],
                                 sem_s[b], add=True)

        fire_idx(0, 0)
        group(0, 0, drain=False, prefetch=True)

        def body(t, _):
            g = 3 * t + 1
            group(g, 1, drain=True, prefetch=True)
            group(g + 1, 2, drain=True, prefetch=True)
            group(g + 2, 0, drain=True, prefetch=True)
            return _

        lax.fori_loop(0, (ng - 2) // 3, body, None)

        group(ng - 1, (ng - 1) % 3, drain=True, prefetch=False)
        drain_s((ng - 1) % 3)

        plsc.subcore_barrier()
        _copy_out(agg_sh, rows_v, agg_out, cid, sid, sem_g)

    return sc_agg


def _make_sc_deg(ng: int):
    num_chunks = ng * NB

    @functools.partial(
        pl.kernel,
        out_type=jax.ShapeDtypeStruct((NC, NP, D), jnp.float32),
        mesh=_MESH,
        scratch_types=(
            [pltpu.VMEM((C,), jnp.int32) for _ in range(3 * NB)]
            + [pltpu.VMEM((C, D), jnp.float32) for _ in range(2)]
            + [pltpu.VMEM((C, D), jnp.float32)]
            + [pltpu.VMEM_SHARED((NP, D), jnp.float32)]
            + [pltpu.SemaphoreType.DMA for _ in range(3 * NB)]
            + [pltpu.SemaphoreType.DMA for _ in range(NB)]
        ),
    )
    def sc_deg(dsts_hbm, zblk_hbm, oblk_hbm, deg_out, *scratch):
        dst_v = scratch[0:3 * NB]
        bounce = scratch[3 * NB:3 * NB + 2]
        ones_v = scratch[3 * NB + 2]
        deg_sh = scratch[3 * NB + 3]
        sem_i = scratch[3 * NB + 4:6 * NB + 4]
        sem_s = scratch[6 * NB + 4:7 * NB + 4]
        cid = lax.axis_index("c")
        sid = lax.axis_index("s")
        wid = cid * NS + sid
        wbase = wid * num_chunks

        _zero_spmem(zblk_hbm, bounce[0], deg_sh, sid, sem_s[0])
        pltpu.sync_copy(oblk_hbm, ones_v)
        plsc.subcore_barrier()

        def fire_idx(g, q):
            for b in range(NB):
                row = wbase + g * NB + b
                pltpu.async_copy(dsts_hbm.at[row], dst_v[q * NB + b],
                                 sem_i[q * NB + b])

        def drain_s(q):
            for b in range(NB):
                pltpu.make_async_copy(ones_v, deg_sh.at[dst_v[q * NB + b]],
                                      sem_s[b]).wait()

        def group(g, q, drain, prefetch):
            if prefetch:
                fire_idx(g + 1, (q + 1) % 3)
            if drain:
                drain_s((q + 2) % 3)
            for b in range(NB):
                i = q * NB + b
                pltpu.make_async_copy(dsts_hbm.at[0], dst_v[i],
                                      sem_i[i]).wait()
                pltpu.async_copy(ones_v, deg_sh.at[dst_v[i]],
                                 sem_s[b], add=True)

        fire_idx(0, 0)
        group(0, 0, drain=False, prefetch=True)

        def body(t, _):
            g = 3 * t + 1
            group(g, 1, drain=True, prefetch=True)
            group(g + 1, 2, drain=True, prefetch=True)
            group(g + 2, 0, drain=True, prefetch=True)
            return _

        lax.fori_loop(0, (ng - 2) // 3, body, None)

        group(ng - 1, (ng - 1) % 3, drain=True, prefetch=False)
        drain_s((ng - 1) % 3)

        plsc.subcore_barrier()
        _copy_out(deg_sh, bounce, deg_out, cid, sid, sem_s)

    return sc_deg


def _make_tc_layer(final: bool, bn: int = 1000):
    def body(x_ref, p_ref, deg_ref, ws_ref, wn_ref, b_ref, o_ref):
        xb = x_ref[...]
        pb = p_ref[0] + p_ref[1]
        deg = deg_ref[0, :, 0:1] + deg_ref[1, :, 0:1]
        hn = pb / jnp.maximum(deg, 1.0)
        acc = lax.dot_general(xb, ws_ref[...], (((1,), (1,)), ((), ())),
                              preferred_element_type=jnp.float32)
        acc = acc + lax.dot_general(hn, wn_ref[...], (((1,), (1,)), ((), ())),
                                    preferred_element_type=jnp.float32)
        acc = acc + b_ref[...]
        acc = jnp.maximum(acc, 0.0)
        if final:
            nrm = jnp.sqrt(jnp.sum(acc * acc, axis=1, keepdims=True))
            acc = acc / jnp.maximum(nrm, 1e-12)
        o_ref[...] = acc

    grid = N // bn
    return pl.pallas_call(
        body,
        grid=(grid,),
        in_specs=[
            pl.BlockSpec((bn, D), lambda i: (i, 0)),
            pl.BlockSpec((NC, bn, D), lambda i: (0, i, 0)),
            pl.BlockSpec((NC, bn, D), lambda i: (0, i, 0)),
            pl.BlockSpec((H, D), lambda i: (0, 0)),
            pl.BlockSpec((H, D), lambda i: (0, 0)),
            pl.BlockSpec((1, H), lambda i: (0, 0)),
        ],
        out_specs=pl.BlockSpec((bn, H), lambda i: (i, 0)),
        out_shape=jax.ShapeDtypeStruct((N, H), jnp.float32),
    )


def kernel(x, edge_index, W_self1, W_neigh1, b1, W_self2, W_neigh2, b2):
    e = edge_index.shape[1]
    ng = _num_groups(e)
    num_chunks = ng * NB
    pad = NW * num_chunks * C - e

    src = edge_index[0]
    dst = edge_index[1]
    pad_idx = jnp.arange(pad, dtype=jnp.int32)
    srcs = jnp.concatenate([src, (pad_idx * 1315) % N]).reshape(
        NW * num_chunks, C)
    dsts = jnp.concatenate([dst, N + pad_idx % 192]).reshape(
        NW * num_chunks, C)

    zblk = jnp.zeros((C, D), jnp.float32)
    oblk = jnp.ones((C, D), jnp.float32)

    sc_agg = _make_sc_agg(ng)
    sc_deg = _make_sc_deg(ng)
    tc_layer1 = _make_tc_layer(final=False)
    tc_layer2 = _make_tc_layer(final=True)

    deg = sc_deg(dsts, zblk, oblk)
    agg1 = sc_agg(x, srcs, dsts, zblk)
    h1 = tc_layer1(x, agg1, deg, W_self1, W_neigh1, b1.reshape(1, H))
    agg2 = sc_agg(h1, srcs, dsts, zblk)
    out = tc_layer2(h1, agg2, deg, W_self2, W_neigh2, b2.reshape(1, H))
    return out
